# baseline (speedup 1.0000x reference)
import jax
import jax.numpy as jnp
from jax.experimental import pallas as pl
from jax.experimental.pallas import tpu as pltpu

_TILE_M = 1024


def _ffn_body(x_ref, w1_ref, b1_ref, w2_any, b2_ref, o_ref,
              w1b_ref, w2f_ref, w2b_ref, w2_sem):
    i = pl.program_id(0)

    @pl.when(i == 0)
    def _start_w2_and_cast_w1():
        pltpu.make_async_copy(w2_any, w2f_ref, w2_sem).start()
        w1b_ref[...] = w1_ref[...].astype(jnp.bfloat16)

    xb = x_ref[...].astype(jnp.bfloat16)
    h = jnp.dot(xb, w1b_ref[...], preferred_element_type=jnp.float32)
    h = jnp.maximum(h + b1_ref[...], 0.0).astype(jnp.bfloat16)

    @pl.when(i == 0)
    def _finish_w2():
        pltpu.make_async_copy(w2_any, w2f_ref, w2_sem).wait()
        w2b_ref[...] = w2f_ref[...].astype(jnp.bfloat16)

    y = jnp.dot(h, w2b_ref[...], preferred_element_type=jnp.float32)
    o_ref[...] = (y + b2_ref[...]).astype(o_ref.dtype)


def _ffn_call(m_rows, tile_m, d_in, d_mid, d_out, out_dtype):
    const = lambda i: (0, 0)
    wkw = {"pipeline_mode": pl.Buffered(1)}
    return pl.pallas_call(
        _ffn_body,
        out_shape=jax.ShapeDtypeStruct((m_rows, d_out), out_dtype),
        grid=(m_rows // tile_m,),
        in_specs=[
            pl.BlockSpec((tile_m, d_in), lambda i: (i, 0)),
            pl.BlockSpec((d_in, d_mid), const, **wkw),
            pl.BlockSpec((1, d_mid), const, **wkw),
            pl.BlockSpec(memory_space=pl.ANY),
            pl.BlockSpec((1, d_out), const, **wkw),
        ],
        out_specs=pl.BlockSpec((tile_m, d_out), lambda i: (i, 0)),
        scratch_shapes=[
            pltpu.VMEM((d_in, d_mid), jnp.bfloat16),
            pltpu.VMEM((d_mid, d_out), jnp.float32),
            pltpu.VMEM((d_mid, d_out), jnp.bfloat16),
            pltpu.SemaphoreType.DMA,
        ],
        compiler_params=pltpu.CompilerParams(
            dimension_semantics=("arbitrary",),
            vmem_limit_bytes=60 * 1024 * 1024,
        ),
    )


@jax.jit
def kernel(x, w1, b1, w2, b2):
    B, S, H = x.shape
    FF = w1.shape[1]
    M = B * S
    x2 = x.reshape(M, H)

    b1f = b1.astype(jnp.float32).reshape(1, FF)
    b2f = b2.astype(jnp.float32).reshape(1, H)

    tile_m = min(_TILE_M, M)
    while M % tile_m:
        tile_m //= 2

    out = _ffn_call(M, tile_m, H, FF, H, x.dtype)(x2, w1, b1f, w2, b2f)
    return out.reshape(B, S, H)


# speedup vs baseline: 1.2837x; 1.2837x over previous
import jax
import jax.numpy as jnp
from jax.experimental import pallas as pl
from jax.experimental.pallas import tpu as pltpu

_TILE_M = 1024


def _ffn_body(x_ref, w1_ref, b1_ref, w2_ref, b2_ref, o_ref, w1b_ref, w2b_ref):
    @pl.when(pl.program_id(0) == 0)
    def _cast_weights():
        w1b_ref[...] = w1_ref[...].astype(jnp.bfloat16)
        w2b_ref[...] = w2_ref[...].astype(jnp.bfloat16)

    xb = x_ref[...].astype(jnp.bfloat16)
    h = jnp.dot(xb, w1b_ref[...], preferred_element_type=jnp.float32)
    h = jnp.maximum(h + b1_ref[...], 0.0).astype(jnp.bfloat16)
    y = jnp.dot(h, w2b_ref[...], preferred_element_type=jnp.float32)
    o_ref[...] = (y + b2_ref[...]).astype(o_ref.dtype)


def _ffn_call(m_rows, tile_m, d_in, d_mid, d_out, out_dtype):
    const = lambda i: (0, 0)
    wkw = {"pipeline_mode": pl.Buffered(1)}
    return pl.pallas_call(
        _ffn_body,
        out_shape=jax.ShapeDtypeStruct((m_rows, d_out), out_dtype),
        grid=(m_rows // tile_m,),
        in_specs=[
            pl.BlockSpec((tile_m, d_in), lambda i: (i, 0)),
            pl.BlockSpec((d_in, d_mid), const, **wkw),
            pl.BlockSpec((1, d_mid), const, **wkw),
            pl.BlockSpec((d_mid, d_out), const, **wkw),
            pl.BlockSpec((1, d_out), const, **wkw),
        ],
        out_specs=pl.BlockSpec((tile_m, d_out), lambda i: (i, 0)),
        scratch_shapes=[
            pltpu.VMEM((d_in, d_mid), jnp.bfloat16),
            pltpu.VMEM((d_mid, d_out), jnp.bfloat16),
        ],
        compiler_params=pltpu.CompilerParams(
            dimension_semantics=("arbitrary",),
            vmem_limit_bytes=60 * 1024 * 1024,
        ),
    )


@jax.jit
def kernel(x, w1, b1, w2, b2):
    B, S, H = x.shape
    FF = w1.shape[1]
    M = B * S
    x2 = x.reshape(M, H)

    b1f = b1.astype(jnp.float32).reshape(1, FF)
    b2f = b2.astype(jnp.float32).reshape(1, H)

    tile_m = min(_TILE_M, M)
    while M % tile_m:
        tile_m //= 2

    out = _ffn_call(M, tile_m, H, FF, H, x.dtype)(x2, w1, b1f, w2, b2f)
    return out.reshape(B, S, H)
